# Initial kernel scaffold; baseline (speedup 1.0000x reference)
import jax
import jax.numpy as jnp
from jax.experimental import pallas as pl


def kernel(x, batch, ppr_idx, ppr_val, W0, W1):
    raise NotImplementedError("write your pallas kernel here")


# speedup vs baseline: 3.2346x; 3.2346x over previous
import functools

import jax
import jax.numpy as jnp
from jax import lax
from jax.experimental import pallas as pl
from jax.experimental.pallas import tpu as pltpu
from jax.experimental.pallas import tpu_sc as plsc

N_NODES = 100000
D_FEAT = 128
HIDDEN = 256
N_CLASSES = 349
BATCH = 16384
TOPK = 32

NUM_CORES = 2
NUM_SUBCORES = 16
LANES = 16
NW = NUM_CORES * NUM_SUBCORES
BPW = BATCH // NW
CHUNK = 128
NCHUNK = BPW // CHUNK
ROWB = 1000
ZROWB = 1024


def _mlp1_body(x_ref, w_ref, o_ref):
    h = jnp.dot(x_ref[...], w_ref[...], preferred_element_type=jnp.float32)
    o_ref[...] = jnp.maximum(h, 0.0)


def _mlp1(x, w0):
    return pl.pallas_call(
        _mlp1_body,
        grid=(N_NODES // ROWB,),
        in_specs=[
            pl.BlockSpec((ROWB, D_FEAT), lambda i: (i, 0)),
            pl.BlockSpec((D_FEAT, HIDDEN), lambda i: (0, 0)),
        ],
        out_specs=pl.BlockSpec((ROWB, HIDDEN), lambda i: (i, 0)),
        out_shape=jax.ShapeDtypeStruct((N_NODES, HIDDEN), jnp.float32),
    )(x, w0)


def _sc_agg_body(h_hbm, batch_hbm, comb_hbm, z_hbm,
                 batch_v, nbwv_v, rows_v, outbuf, gsem):
    wid = lax.axis_index("s") * NUM_CORES + lax.axis_index("c")
    base = wid * BPW

    pltpu.sync_copy(batch_hbm.at[pl.ds(wid * (BPW // 128), BPW // 128)], batch_v)

    cps = []
    for c in range(BPW // 128):
        cps.append(pltpu.async_copy(
            comb_hbm.at[batch_v.at[c]], nbwv_v.at[pl.ds(c * 128, 128)], gsem))
    for cp in cps:
        cp.wait()

    def item(bl, chunk):
        b = chunk * CHUNK + bl
        pltpu.async_copy(h_hbm.at[nbwv_v.at[b, pl.ds(0, TOPK)]], rows_v, gsem).wait()
        accs = [jnp.zeros((LANES,), jnp.float32) for _ in range(HIDDEN // LANES)]
        bb = jnp.full((LANES,), b, jnp.int32)
        for k in range(TOPK):
            wbits = plsc.load_gather(
                nbwv_v, [bb, jnp.full((LANES,), TOPK + k, jnp.int32)])
            w = plsc.bitcast(wbits, jnp.float32)
            for j in range(HIDDEN // LANES):
                accs[j] = accs[j] + w * rows_v[k, pl.ds(j * LANES, LANES)]
        for j in range(HIDDEN // LANES):
            outbuf[bl, pl.ds(j * LANES, LANES)] = accs[j]
        return chunk

    def chunk_body(chunk, carry):
        lax.fori_loop(0, CHUNK, item, chunk)
        pltpu.sync_copy(outbuf, z_hbm.at[pl.ds(base + chunk * CHUNK, CHUNK)])
        return carry

    lax.fori_loop(0, NCHUNK, chunk_body, 0)


_sc_agg = functools.partial(
    pl.kernel,
    out_type=jax.ShapeDtypeStruct((BATCH, HIDDEN), jnp.float32),
    mesh=plsc.VectorSubcoreMesh(
        core_axis_name="c", subcore_axis_name="s",
        num_cores=NUM_CORES, num_subcores=NUM_SUBCORES),
    scratch_types=[
        pltpu.VMEM((BPW // 128, 128), jnp.int32),
        pltpu.VMEM((BPW, 128), jnp.int32),
        pltpu.VMEM((TOPK, HIDDEN), jnp.float32),
        pltpu.VMEM((CHUNK, HIDDEN), jnp.float32),
        pltpu.SemaphoreType.DMA,
    ],
    compiler_params=pltpu.CompilerParams(needs_layout_passes=False),
)(_sc_agg_body)


def _mlp2_body(z_ref, w_ref, o_ref):
    logits = jnp.dot(z_ref[...], w_ref[...], preferred_element_type=jnp.float32)
    m = jnp.max(logits, axis=1, keepdims=True)
    e = jnp.exp(logits - m)
    s = jnp.sum(e, axis=1, keepdims=True)
    o_ref[...] = (logits - m) - jnp.log(s)


def _mlp2(z, w1):
    return pl.pallas_call(
        _mlp2_body,
        grid=(BATCH // ZROWB,),
        in_specs=[
            pl.BlockSpec((ZROWB, HIDDEN), lambda i: (i, 0)),
            pl.BlockSpec((HIDDEN, N_CLASSES), lambda i: (0, 0)),
        ],
        out_specs=pl.BlockSpec((ZROWB, N_CLASSES), lambda i: (i, 0)),
        out_shape=jax.ShapeDtypeStruct((BATCH, N_CLASSES), jnp.float32),
    )(z, w1)


def kernel(x, batch, ppr_idx, ppr_val, W0, W1):
    batch2d = batch.astype(jnp.int32).reshape(BATCH // 128, 128)
    comb = jnp.concatenate([
        ppr_idx.astype(jnp.int32),
        jax.lax.bitcast_convert_type(ppr_val, jnp.int32),
        jnp.zeros((N_NODES, 128 - 2 * TOPK), jnp.int32),
    ], axis=1)
    h = _mlp1(x, W0)
    z = _sc_agg(h, batch2d, comb)
    return _mlp2(z, W1)


# speedup vs baseline: 9.1315x; 2.8231x over previous
import functools

import jax
import jax.numpy as jnp
from jax import lax
from jax.experimental import pallas as pl
from jax.experimental.pallas import tpu as pltpu
from jax.experimental.pallas import tpu_sc as plsc

N_NODES = 100000
D_FEAT = 128
HIDDEN = 256
N_CLASSES = 349
BATCH = 16384
TOPK = 32

NUM_CORES = 2
NUM_SUBCORES = 16
LANES = 16
NW = NUM_CORES * NUM_SUBCORES
BPW = BATCH // NW
CHUNK = 64
ROWB = 1000
ZROWB = 1024


def _mlp1_body(x_ref, w_ref, o_ref):
    h = jnp.dot(x_ref[...], w_ref[...], preferred_element_type=jnp.float32)
    o_ref[...] = jnp.maximum(h, 0.0)


def _mlp1(x, w0):
    return pl.pallas_call(
        _mlp1_body,
        grid=(N_NODES // ROWB,),
        in_specs=[
            pl.BlockSpec((ROWB, D_FEAT), lambda i: (i, 0)),
            pl.BlockSpec((D_FEAT, HIDDEN), lambda i: (0, 0)),
        ],
        out_specs=pl.BlockSpec((ROWB, HIDDEN), lambda i: (i, 0)),
        out_shape=jax.ShapeDtypeStruct((N_NODES, HIDDEN), jnp.float32),
    )(x, w0)


NBUF = 4


def _sc_agg_body(h_hbm, batch_hbm, comb_hbm, z_hbm,
                 batch_v, nbwv_v, rows_v, outbuf, gsem, hsem):
    wid = lax.axis_index("s") * NUM_CORES + lax.axis_index("c")
    base = wid * BPW

    pltpu.sync_copy(batch_hbm.at[pl.ds(wid * (BPW // 128), BPW // 128)], batch_v)

    cps = []
    for c in range(BPW // 128):
        cps.append(pltpu.async_copy(
            comb_hbm.at[batch_v.at[c]], nbwv_v.at[pl.ds(c * 128, 128)], gsem))
    for cp in cps:
        cp.wait()

    def fire(b):
        pltpu.async_copy(
            h_hbm.at[nbwv_v.at[b, pl.ds(0, TOPK)]],
            rows_v.at[b & (NBUF - 1)], hsem)

    def drain_one(b):
        pltpu.make_async_copy(
            h_hbm.at[nbwv_v.at[b, pl.ds(0, TOPK)]],
            rows_v.at[b & (NBUF - 1)], hsem).wait()

    for p in range(NBUF - 1):
        fire(jnp.int32(p))

    def item(b, carry):
        @pl.when(b + (NBUF - 1) < BPW)
        def _():
            fire(b + (NBUF - 1))
        drain_one(b)
        buf = b & (NBUF - 1)
        accs = [jnp.zeros((LANES,), jnp.float32) for _ in range(HIDDEN // LANES)]
        bb = jnp.full((LANES,), b, jnp.int32)
        for k in range(TOPK):
            wbits = plsc.load_gather(
                nbwv_v, [bb, jnp.full((LANES,), TOPK + k, jnp.int32)])
            w = plsc.bitcast(wbits, jnp.float32)
            for j in range(HIDDEN // LANES):
                accs[j] = accs[j] + w * rows_v[buf, k, pl.ds(j * LANES, LANES)]
        bl = b & (CHUNK - 1)
        for j in range(HIDDEN // LANES):
            outbuf[bl, pl.ds(j * LANES, LANES)] = accs[j]

        @pl.when(bl == CHUNK - 1)
        def _():
            start = pl.multiple_of(base + (b - (CHUNK - 1)), CHUNK)
            pltpu.sync_copy(outbuf, z_hbm.at[pl.ds(start, CHUNK)])
        return carry

    lax.fori_loop(0, BPW, item, 0)


_sc_agg = functools.partial(
    pl.kernel,
    out_type=jax.ShapeDtypeStruct((BATCH, HIDDEN), jnp.float32),
    mesh=plsc.VectorSubcoreMesh(
        core_axis_name="c", subcore_axis_name="s",
        num_cores=NUM_CORES, num_subcores=NUM_SUBCORES),
    scratch_types=[
        pltpu.VMEM((BPW // 128, 128), jnp.int32),
        pltpu.VMEM((BPW, 128), jnp.int32),
        pltpu.VMEM((NBUF, TOPK, HIDDEN), jnp.float32),
        pltpu.VMEM((CHUNK, HIDDEN), jnp.float32),
        pltpu.SemaphoreType.DMA,
        pltpu.SemaphoreType.DMA,
    ],
    compiler_params=pltpu.CompilerParams(needs_layout_passes=False),
)(_sc_agg_body)


def _mlp2_body(z_ref, w_ref, o_ref):
    logits = jnp.dot(z_ref[...], w_ref[...], preferred_element_type=jnp.float32)
    m = jnp.max(logits, axis=1, keepdims=True)
    e = jnp.exp(logits - m)
    s = jnp.sum(e, axis=1, keepdims=True)
    o_ref[...] = (logits - m) - jnp.log(s)


def _mlp2(z, w1):
    return pl.pallas_call(
        _mlp2_body,
        grid=(BATCH // ZROWB,),
        in_specs=[
            pl.BlockSpec((ZROWB, HIDDEN), lambda i: (i, 0)),
            pl.BlockSpec((HIDDEN, N_CLASSES), lambda i: (0, 0)),
        ],
        out_specs=pl.BlockSpec((ZROWB, N_CLASSES), lambda i: (i, 0)),
        out_shape=jax.ShapeDtypeStruct((BATCH, N_CLASSES), jnp.float32),
    )(z, w1)


def kernel(x, batch, ppr_idx, ppr_val, W0, W1):
    batch2d = batch.astype(jnp.int32).reshape(BATCH // 128, 128)
    comb = jnp.concatenate([
        ppr_idx.astype(jnp.int32),
        jax.lax.bitcast_convert_type(ppr_val, jnp.int32),
        jnp.zeros((N_NODES, 128 - 2 * TOPK), jnp.int32),
    ], axis=1)
    h = _mlp1(x, W0)
    z = _sc_agg(h, batch2d, comb)
    return _mlp2(z, W1)


# speedup vs baseline: 13.0388x; 1.4279x over previous
import functools

import jax
import jax.numpy as jnp
import numpy as np
from jax import lax
from jax.experimental import pallas as pl
from jax.experimental.pallas import tpu as pltpu
from jax.experimental.pallas import tpu_sc as plsc

N_NODES = 100000
D_FEAT = 128
HIDDEN = 256
N_CLASSES = 349
BATCH = 16384
TOPK = 32

NUM_CORES = 2
NUM_SUBCORES = 16
LANES = 16
NW = NUM_CORES * NUM_SUBCORES
BPW = BATCH // NW
CHUNK = 64
ROWB = 1000
ZROWB = 1024


def _mlp1_body(x_ref, w_ref, o_ref):
    h = jnp.dot(x_ref[...], w_ref[...], preferred_element_type=jnp.float32)
    h = jnp.maximum(h, 0.0)
    ia = jax.lax.bitcast_convert_type(h[:, : HIDDEN // 2], jnp.int32)
    ib = jax.lax.bitcast_convert_type(h[:, HIDDEN // 2:], jnp.int32)
    lo = jax.lax.shift_right_logical(ia + 0x8000, 16)
    hi = (ib + 0x8000) & jnp.int32(-65536)
    o_ref[...] = lo | hi


def _mlp1(x, w0):
    return pl.pallas_call(
        _mlp1_body,
        grid=(N_NODES // ROWB,),
        in_specs=[
            pl.BlockSpec((ROWB, D_FEAT), lambda i: (i, 0)),
            pl.BlockSpec((D_FEAT, HIDDEN), lambda i: (0, 0)),
        ],
        out_specs=pl.BlockSpec((ROWB, HIDDEN // 2), lambda i: (i, 0)),
        out_shape=jax.ShapeDtypeStruct((N_NODES, HIDDEN // 2), jnp.int32),
    )(x, w0)


NBUF = 4


def _sc_agg_body(h_hbm, batch_hbm, comb_hbm, z_hbm,
                 batch_v, nbwv_v, rows_v, outbuf, gsem, hsem):
    wid = lax.axis_index("s") * NUM_CORES + lax.axis_index("c")
    base = wid * BPW

    pltpu.sync_copy(batch_hbm.at[pl.ds(wid * (BPW // 128), BPW // 128)], batch_v)

    cps = []
    for c in range(BPW // 128):
        cps.append(pltpu.async_copy(
            comb_hbm.at[batch_v.at[c]], nbwv_v.at[pl.ds(c * 128, 128)], gsem))
    for cp in cps:
        cp.wait()

    def fire(b):
        pltpu.async_copy(
            h_hbm.at[nbwv_v.at[b, pl.ds(0, TOPK)]],
            rows_v.at[b & (NBUF - 1)], hsem)

    def drain_one(b):
        pltpu.make_async_copy(
            h_hbm.at[nbwv_v.at[b, pl.ds(0, TOPK)]],
            rows_v.at[b & (NBUF - 1)], hsem).wait()

    for p in range(NBUF - 1):
        fire(jnp.int32(p))

    def item(b, carry):
        @pl.when(b + (NBUF - 1) < BPW)
        def _():
            fire(b + (NBUF - 1))
        drain_one(b)
        buf = b & (NBUF - 1)
        nj = HIDDEN // (2 * LANES)
        accs = [jnp.zeros((LANES,), jnp.float32) for _ in range(HIDDEN // LANES)]
        bb = jnp.full((LANES,), b, jnp.int32)
        for k in range(TOPK):
            wbits = plsc.load_gather(
                nbwv_v, [bb, jnp.full((LANES,), TOPK + k, jnp.int32)])
            w = plsc.bitcast(wbits, jnp.float32)
            for j in range(nj):
                v = rows_v[buf, k, pl.ds(j * LANES, LANES)]
                lo = plsc.bitcast(jax.lax.shift_left(v, 16), jnp.float32)
                hi = plsc.bitcast(v & jnp.int32(-65536), jnp.float32)
                accs[j] = accs[j] + w * lo
                accs[nj + j] = accs[nj + j] + w * hi
        bl = b & (CHUNK - 1)
        for j in range(HIDDEN // LANES):
            outbuf[bl, pl.ds(j * LANES, LANES)] = accs[j]

        @pl.when(bl == CHUNK - 1)
        def _():
            start = pl.multiple_of(base + (b - (CHUNK - 1)), CHUNK)
            pltpu.sync_copy(outbuf, z_hbm.at[pl.ds(start, CHUNK)])
        return carry

    lax.fori_loop(0, BPW, item, 0)


_sc_agg = functools.partial(
    pl.kernel,
    out_type=jax.ShapeDtypeStruct((BATCH, HIDDEN), jnp.float32),
    mesh=plsc.VectorSubcoreMesh(
        core_axis_name="c", subcore_axis_name="s",
        num_cores=NUM_CORES, num_subcores=NUM_SUBCORES),
    scratch_types=[
        pltpu.VMEM((BPW // 128, 128), jnp.int32),
        pltpu.VMEM((BPW, 128), jnp.int32),
        pltpu.VMEM((NBUF, TOPK, HIDDEN // 2), jnp.int32),
        pltpu.VMEM((CHUNK, HIDDEN), jnp.float32),
        pltpu.SemaphoreType.DMA,
        pltpu.SemaphoreType.DMA,
    ],
    compiler_params=pltpu.CompilerParams(needs_layout_passes=False),
)(_sc_agg_body)


def _mlp2_body(z_ref, w_ref, o_ref):
    logits = jnp.dot(z_ref[...], w_ref[...], preferred_element_type=jnp.float32)
    m = jnp.max(logits, axis=1, keepdims=True)
    e = jnp.exp(logits - m)
    s = jnp.sum(e, axis=1, keepdims=True)
    o_ref[...] = (logits - m) - jnp.log(s)


def _mlp2(z, w1):
    return pl.pallas_call(
        _mlp2_body,
        grid=(BATCH // ZROWB,),
        in_specs=[
            pl.BlockSpec((ZROWB, HIDDEN), lambda i: (i, 0)),
            pl.BlockSpec((HIDDEN, N_CLASSES), lambda i: (0, 0)),
        ],
        out_specs=pl.BlockSpec((ZROWB, N_CLASSES), lambda i: (i, 0)),
        out_shape=jax.ShapeDtypeStruct((BATCH, N_CLASSES), jnp.float32),
    )(z, w1)


def kernel(x, batch, ppr_idx, ppr_val, W0, W1):
    batch2d = batch.astype(jnp.int32).reshape(BATCH // 128, 128)
    comb = jnp.concatenate([
        ppr_idx.astype(jnp.int32),
        jax.lax.bitcast_convert_type(ppr_val, jnp.int32),
        jnp.zeros((N_NODES, 128 - 2 * TOPK), jnp.int32),
    ], axis=1)
    h = _mlp1(x, W0)
    z = _sc_agg(h, batch2d, comb)
    return _mlp2(z, W1)


# speedup vs baseline: 13.1852x; 1.0112x over previous
import functools

import jax
import jax.numpy as jnp
import numpy as np
from jax import lax
from jax.experimental import pallas as pl
from jax.experimental.pallas import tpu as pltpu
from jax.experimental.pallas import tpu_sc as plsc

N_NODES = 100000
D_FEAT = 128
HIDDEN = 256
N_CLASSES = 349
BATCH = 16384
TOPK = 32

NUM_CORES = 2
NUM_SUBCORES = 16
LANES = 16
NW = NUM_CORES * NUM_SUBCORES
BPW = BATCH // NW
CHUNK = 32
LOGC = 5
ROWB = 1000
ZROWB = 1024


def _mlp1_body(x_ref, w_ref, idx_ref, val_ref, o_ref, comb_ref):
    xb = x_ref[...].astype(jnp.bfloat16)
    wb = w_ref[...].astype(jnp.bfloat16)
    h = jnp.dot(xb, wb, preferred_element_type=jnp.float32)
    h = jnp.maximum(h, 0.0)
    ia = jax.lax.bitcast_convert_type(h[:, : HIDDEN // 2], jnp.int32)
    ib = jax.lax.bitcast_convert_type(h[:, HIDDEN // 2:], jnp.int32)
    lo = jax.lax.shift_right_logical(ia + 0x8000, 16)
    hi = ib & jnp.int32(-65536)
    o_ref[...] = lo | hi
    comb_ref[...] = jnp.concatenate([
        idx_ref[...],
        jax.lax.bitcast_convert_type(val_ref[...], jnp.int32),
        jnp.zeros((ROWB, 128 - 2 * TOPK), jnp.int32),
    ], axis=1)


def _mlp1(x, w0, ppr_idx, ppr_val):
    return pl.pallas_call(
        _mlp1_body,
        grid=(N_NODES // ROWB,),
        in_specs=[
            pl.BlockSpec((ROWB, D_FEAT), lambda i: (i, 0)),
            pl.BlockSpec((D_FEAT, HIDDEN), lambda i: (0, 0)),
            pl.BlockSpec((ROWB, TOPK), lambda i: (i, 0)),
            pl.BlockSpec((ROWB, TOPK), lambda i: (i, 0)),
        ],
        out_specs=[
            pl.BlockSpec((ROWB, HIDDEN // 2), lambda i: (i, 0)),
            pl.BlockSpec((ROWB, 128), lambda i: (i, 0)),
        ],
        out_shape=[
            jax.ShapeDtypeStruct((N_NODES, HIDDEN // 2), jnp.int32),
            jax.ShapeDtypeStruct((N_NODES, 128), jnp.int32),
        ],
    )(x, w0, ppr_idx, ppr_val)


NBUF = 8


def _sc_agg_body(h_hbm, batch_hbm, comb_hbm, z_hbm,
                 batch_v, nbwv_v, rows_v, outbuf, gsem, hsem, osem):
    wid = lax.axis_index("s") * NUM_CORES + lax.axis_index("c")
    base = wid * BPW

    pltpu.sync_copy(batch_hbm.at[pl.ds(wid * (BPW // 128), BPW // 128)], batch_v)

    cps = []
    for c in range(BPW // 128):
        cps.append(pltpu.async_copy(
            comb_hbm.at[batch_v.at[c]], nbwv_v.at[pl.ds(c * 128, 128)], gsem))
    for cp in cps:
        cp.wait()

    def fire(b):
        pltpu.async_copy(
            h_hbm.at[nbwv_v.at[b, pl.ds(0, TOPK)]],
            rows_v.at[b & (NBUF - 1)], hsem)

    def drain_one(b):
        pltpu.make_async_copy(
            h_hbm.at[nbwv_v.at[b, pl.ds(0, TOPK)]],
            rows_v.at[b & (NBUF - 1)], hsem).wait()

    for p in range(NBUF - 1):
        fire(jnp.int32(p))

    def item(b, carry):
        @pl.when(b + (NBUF - 1) < BPW)
        def _():
            fire(b + (NBUF - 1))
        drain_one(b)
        buf = b & (NBUF - 1)
        bl = b & (CHUNK - 1)
        half = (b >> LOGC) & 1

        @pl.when((bl == 0) & (b >= 2 * CHUNK))
        def _():
            pltpu.make_async_copy(
                outbuf.at[half], z_hbm.at[pl.ds(base, CHUNK)], osem).wait()

        nj = HIDDEN // (2 * LANES)
        accs = [jnp.zeros((LANES,), jnp.float32) for _ in range(HIDDEN // LANES)]
        bb = jnp.full((LANES,), b, jnp.int32)
        for k in range(TOPK):
            wbits = plsc.load_gather(
                nbwv_v, [bb, jnp.full((LANES,), TOPK + k, jnp.int32)])
            w = plsc.bitcast(wbits, jnp.float32)
            for j in range(nj):
                v = rows_v[buf, k, pl.ds(j * LANES, LANES)]
                lo = plsc.bitcast(jax.lax.shift_left(v, 16), jnp.float32)
                hi = plsc.bitcast(v, jnp.float32)
                accs[j] = accs[j] + w * lo
                accs[nj + j] = accs[nj + j] + w * hi
        for j in range(HIDDEN // LANES):
            outbuf[half, bl, pl.ds(j * LANES, LANES)] = accs[j]

        @pl.when(bl == CHUNK - 1)
        def _():
            start = pl.multiple_of(base + (b - (CHUNK - 1)), CHUNK)
            pltpu.async_copy(outbuf.at[half], z_hbm.at[pl.ds(start, CHUNK)], osem)
        return carry

    lax.fori_loop(0, BPW, item, 0)
    for _ in range(2):
        pltpu.make_async_copy(
            outbuf.at[0], z_hbm.at[pl.ds(base, CHUNK)], osem).wait()


_sc_agg = functools.partial(
    pl.kernel,
    out_type=jax.ShapeDtypeStruct((BATCH, HIDDEN), jnp.float32),
    mesh=plsc.VectorSubcoreMesh(
        core_axis_name="c", subcore_axis_name="s",
        num_cores=NUM_CORES, num_subcores=NUM_SUBCORES),
    scratch_types=[
        pltpu.VMEM((BPW // 128, 128), jnp.int32),
        pltpu.VMEM((BPW, 128), jnp.int32),
        pltpu.VMEM((NBUF, TOPK, HIDDEN // 2), jnp.int32),
        pltpu.VMEM((2, CHUNK, HIDDEN), jnp.float32),
        pltpu.SemaphoreType.DMA,
        pltpu.SemaphoreType.DMA,
        pltpu.SemaphoreType.DMA,
    ],
    compiler_params=pltpu.CompilerParams(needs_layout_passes=False),
)(_sc_agg_body)


def _mlp2_body(z_ref, w_ref, o_ref):
    logits = jnp.dot(z_ref[...], w_ref[...], preferred_element_type=jnp.float32)
    m = jnp.max(logits, axis=1, keepdims=True)
    e = jnp.exp(logits - m)
    s = jnp.sum(e, axis=1, keepdims=True)
    o_ref[...] = (logits - m) - jnp.log(s)


def _mlp2(z, w1):
    return pl.pallas_call(
        _mlp2_body,
        grid=(BATCH // ZROWB,),
        in_specs=[
            pl.BlockSpec((ZROWB, HIDDEN), lambda i: (i, 0)),
            pl.BlockSpec((HIDDEN, N_CLASSES), lambda i: (0, 0)),
        ],
        out_specs=pl.BlockSpec((ZROWB, N_CLASSES), lambda i: (i, 0)),
        out_shape=jax.ShapeDtypeStruct((BATCH, N_CLASSES), jnp.float32),
    )(z, w1)


def kernel(x, batch, ppr_idx, ppr_val, W0, W1):
    batch2d = batch.astype(jnp.int32).reshape(BATCH // 128, 128)
    h, comb = _mlp1(x, W0, ppr_idx.astype(jnp.int32), ppr_val)
    z = _sc_agg(h, batch2d, comb)
    return _mlp2(z, W1)


# speedup vs baseline: 16.7208x; 1.2681x over previous
import functools

import jax
import jax.numpy as jnp
import numpy as np
from jax import lax
from jax.experimental import pallas as pl
from jax.experimental.pallas import tpu as pltpu
from jax.experimental.pallas import tpu_sc as plsc

N_NODES = 100000
D_FEAT = 128
HIDDEN = 256
N_CLASSES = 349
BATCH = 16384
TOPK = 32

NUM_CORES = 2
NUM_SUBCORES = 16
LANES = 16
NW = NUM_CORES * NUM_SUBCORES
BPW = BATCH // NW
CHUNK = 32
LOGC = 5
ROWB = 1024
ZROWB = 512


def _mlp1_body(x_ref, w_ref, idxt_ref, valt_ref, o_ref, comb_ref):
    xb = x_ref[...].astype(jnp.bfloat16)
    wb = w_ref[...].astype(jnp.bfloat16)
    h = jnp.dot(xb, wb, preferred_element_type=jnp.float32)
    h = jnp.maximum(h, 0.0)
    ia = jax.lax.bitcast_convert_type(h[:, : HIDDEN // 2], jnp.int32)
    ib = jax.lax.bitcast_convert_type(h[:, HIDDEN // 2:], jnp.int32)
    lo = jax.lax.shift_right_logical(ia + 0x8000, 16)
    hi = ib & jnp.int32(-65536)
    o_ref[...] = lo | hi
    comb_ref[...] = jnp.concatenate([
        jnp.transpose(idxt_ref[...]),
        jax.lax.bitcast_convert_type(jnp.transpose(valt_ref[...]), jnp.int32),
        jnp.zeros((ROWB, 128 - 2 * TOPK), jnp.int32),
    ], axis=1)


def _mlp1(x, w0, ppr_idx_t, ppr_val_t):
    return pl.pallas_call(
        _mlp1_body,
        grid=(pl.cdiv(N_NODES, ROWB),),
        in_specs=[
            pl.BlockSpec((ROWB, D_FEAT), lambda i: (i, 0)),
            pl.BlockSpec((D_FEAT, HIDDEN), lambda i: (0, 0)),
            pl.BlockSpec((TOPK, ROWB), lambda i: (0, i)),
            pl.BlockSpec((TOPK, ROWB), lambda i: (0, i)),
        ],
        out_specs=[
            pl.BlockSpec((ROWB, HIDDEN // 2), lambda i: (i, 0)),
            pl.BlockSpec((ROWB, 128), lambda i: (i, 0)),
        ],
        out_shape=[
            jax.ShapeDtypeStruct((N_NODES, HIDDEN // 2), jnp.int32),
            jax.ShapeDtypeStruct((N_NODES, 128), jnp.int32),
        ],
    )(x, w0, ppr_idx_t, ppr_val_t)


NBUF = 8


def _sc_agg_body(h_hbm, batch_hbm, comb_hbm, z_hbm,
                 batch_v, nbwv_v, rows_v, outbuf, gsem, hsem, osem):
    wid = lax.axis_index("s") * NUM_CORES + lax.axis_index("c")
    base = wid * BPW

    pltpu.sync_copy(batch_hbm.at[pl.ds(wid * (BPW // 128), BPW // 128)], batch_v)

    cps = []
    for c in range(BPW // 128):
        cps.append(pltpu.async_copy(
            comb_hbm.at[batch_v.at[c]], nbwv_v.at[pl.ds(c * 128, 128)], gsem))
    for cp in cps:
        cp.wait()

    def fire(b):
        pltpu.async_copy(
            h_hbm.at[nbwv_v.at[b, pl.ds(0, TOPK)]],
            rows_v.at[b & (NBUF - 1)], hsem)

    def drain_one(b):
        pltpu.make_async_copy(
            h_hbm.at[nbwv_v.at[b, pl.ds(0, TOPK)]],
            rows_v.at[b & (NBUF - 1)], hsem).wait()

    for p in range(NBUF - 1):
        fire(jnp.int32(p))

    def item(b, carry):
        @pl.when(b + (NBUF - 1) < BPW)
        def _():
            fire(b + (NBUF - 1))
        drain_one(b)
        buf = b & (NBUF - 1)
        bl = b & (CHUNK - 1)
        half = (b >> LOGC) & 1

        @pl.when((bl == 0) & (b >= 2 * CHUNK))
        def _():
            pltpu.make_async_copy(
                outbuf.at[half], z_hbm.at[pl.ds(base, CHUNK)], osem).wait()

        nj = HIDDEN // (2 * LANES)
        accs = [jnp.zeros((LANES,), jnp.float32) for _ in range(HIDDEN // LANES)]
        bb = jnp.full((LANES,), b, jnp.int32)
        for k in range(TOPK):
            wbits = plsc.load_gather(
                nbwv_v, [bb, jnp.full((LANES,), TOPK + k, jnp.int32)])
            w = plsc.bitcast(wbits, jnp.float32)
            for j in range(nj):
                v = rows_v[buf, k, pl.ds(j * LANES, LANES)]
                lo = plsc.bitcast(jax.lax.shift_left(v, 16), jnp.float32)
                hi = plsc.bitcast(v, jnp.float32)
                accs[j] = accs[j] + w * lo
                accs[nj + j] = accs[nj + j] + w * hi
        for j in range(HIDDEN // LANES):
            outbuf[half, bl, pl.ds(j * LANES, LANES)] = accs[j]

        @pl.when(bl == CHUNK - 1)
        def _():
            start = pl.multiple_of(base + (b - (CHUNK - 1)), CHUNK)
            pltpu.async_copy(outbuf.at[half], z_hbm.at[pl.ds(start, CHUNK)], osem)
        return carry

    lax.fori_loop(0, BPW, item, 0)
    for _ in range(2):
        pltpu.make_async_copy(
            outbuf.at[0], z_hbm.at[pl.ds(base, CHUNK)], osem).wait()


_sc_agg = functools.partial(
    pl.kernel,
    out_type=jax.ShapeDtypeStruct((BATCH, HIDDEN), jnp.float32),
    mesh=plsc.VectorSubcoreMesh(
        core_axis_name="c", subcore_axis_name="s",
        num_cores=NUM_CORES, num_subcores=NUM_SUBCORES),
    scratch_types=[
        pltpu.VMEM((BPW // 128, 128), jnp.int32),
        pltpu.VMEM((BPW, 128), jnp.int32),
        pltpu.VMEM((NBUF, TOPK, HIDDEN // 2), jnp.int32),
        pltpu.VMEM((2, CHUNK, HIDDEN), jnp.float32),
        pltpu.SemaphoreType.DMA,
        pltpu.SemaphoreType.DMA,
        pltpu.SemaphoreType.DMA,
    ],
    compiler_params=pltpu.CompilerParams(needs_layout_passes=False),
)(_sc_agg_body)


def _mlp2_body(z_ref, w_ref, o_ref):
    logits = jax.lax.dot_general(
        w_ref[...], z_ref[...], (((0,), (1,)), ((), ())),
        preferred_element_type=jnp.float32)
    m = jnp.max(logits, axis=0, keepdims=True)
    e = jnp.exp(logits - m)
    s = jnp.sum(e, axis=0, keepdims=True)
    o_ref[...] = (logits - m) - jnp.log(s)


def _mlp2(z, w1):
    return pl.pallas_call(
        _mlp2_body,
        grid=(BATCH // ZROWB,),
        in_specs=[
            pl.BlockSpec((ZROWB, HIDDEN), lambda i: (i, 0)),
            pl.BlockSpec((HIDDEN, N_CLASSES), lambda i: (0, 0)),
        ],
        out_specs=pl.BlockSpec((N_CLASSES, ZROWB), lambda i: (0, i)),
        out_shape=jax.ShapeDtypeStruct((N_CLASSES, BATCH), jnp.float32),
    )(z, w1)


def kernel(x, batch, ppr_idx, ppr_val, W0, W1):
    batch2d = batch.astype(jnp.int32).reshape(BATCH // 128, 128)
    h, comb = _mlp1(x, W0, ppr_idx.astype(jnp.int32).T, ppr_val.T)
    z = _sc_agg(h, batch2d, comb)
    return _mlp2(z, W1).T


# speedup vs baseline: 17.1268x; 1.0243x over previous
import functools

import jax
import jax.numpy as jnp
import numpy as np
from jax import lax
from jax.experimental import pallas as pl
from jax.experimental.pallas import tpu as pltpu
from jax.experimental.pallas import tpu_sc as plsc

N_NODES = 100000
D_FEAT = 128
HIDDEN = 256
N_CLASSES = 349
BATCH = 16384
TOPK = 32

NUM_CORES = 2
NUM_SUBCORES = 16
LANES = 16
NW = NUM_CORES * NUM_SUBCORES
BPW = BATCH // NW
CHUNK = 32
LOGC = 5
ROWB = 1024
ZROWB = 1024


def _mlp1_body(x_ref, w_ref, idxt_ref, valt_ref, o_ref, comb_ref):
    xb = x_ref[...].astype(jnp.bfloat16)
    wb = w_ref[...].astype(jnp.bfloat16)
    h = jnp.dot(xb, wb, preferred_element_type=jnp.float32)
    h = jnp.maximum(h, 0.0)
    ia = jax.lax.bitcast_convert_type(h[:, : HIDDEN // 2], jnp.int32)
    ib = jax.lax.bitcast_convert_type(h[:, HIDDEN // 2:], jnp.int32)
    lo = jax.lax.shift_right_logical(ia + 0x8000, 16)
    hi = ib & jnp.int32(-65536)
    o_ref[...] = lo | hi
    comb_ref[...] = jnp.concatenate([
        jnp.transpose(idxt_ref[...]),
        jax.lax.bitcast_convert_type(jnp.transpose(valt_ref[...]), jnp.int32),
        jnp.zeros((ROWB, 128 - 2 * TOPK), jnp.int32),
    ], axis=1)


def _mlp1(x, w0, ppr_idx_t, ppr_val_t):
    return pl.pallas_call(
        _mlp1_body,
        grid=(pl.cdiv(N_NODES, ROWB),),
        in_specs=[
            pl.BlockSpec((ROWB, D_FEAT), lambda i: (i, 0)),
            pl.BlockSpec((D_FEAT, HIDDEN), lambda i: (0, 0)),
            pl.BlockSpec((TOPK, ROWB), lambda i: (0, i)),
            pl.BlockSpec((TOPK, ROWB), lambda i: (0, i)),
        ],
        out_specs=[
            pl.BlockSpec((ROWB, HIDDEN // 2), lambda i: (i, 0)),
            pl.BlockSpec((ROWB, 128), lambda i: (i, 0)),
        ],
        out_shape=[
            jax.ShapeDtypeStruct((N_NODES, HIDDEN // 2), jnp.int32),
            jax.ShapeDtypeStruct((N_NODES, 128), jnp.int32),
        ],
    )(x, w0, ppr_idx_t, ppr_val_t)


NBUF = 16


def _sc_agg_body(h_hbm, batch_hbm, comb_hbm, z_hbm,
                 batch_v, nbwv_v, rows_v, outbuf, gsem, hsem, osem):
    wid = lax.axis_index("s") * NUM_CORES + lax.axis_index("c")
    base = wid * BPW

    pltpu.sync_copy(batch_hbm.at[pl.ds(wid * (BPW // 128), BPW // 128)], batch_v)

    cps = []
    for c in range(BPW // 128):
        cps.append(pltpu.async_copy(
            comb_hbm.at[batch_v.at[c]], nbwv_v.at[pl.ds(c * 128, 128)], gsem))
    for cp in cps:
        cp.wait()

    HALFK = TOPK // 2

    def fire(b):
        s0 = (2 * b) & (NBUF - 1)
        pltpu.async_copy(
            h_hbm.at[nbwv_v.at[b, pl.ds(0, HALFK)]], rows_v.at[s0], hsem)
        pltpu.async_copy(
            h_hbm.at[nbwv_v.at[b, pl.ds(HALFK, HALFK)]], rows_v.at[s0 + 1], hsem)

    def drain_one(b):
        s0 = (2 * b) & (NBUF - 1)
        pltpu.make_async_copy(
            h_hbm.at[nbwv_v.at[b, pl.ds(0, HALFK)]], rows_v.at[s0], hsem).wait()
        pltpu.make_async_copy(
            h_hbm.at[nbwv_v.at[b, pl.ds(0, HALFK)]], rows_v.at[s0 + 1], hsem).wait()

    AHEAD = NBUF // 2 - 1
    for p in range(AHEAD):
        fire(jnp.int32(p))

    def item(b, carry):
        @pl.when(b + AHEAD < BPW)
        def _():
            fire(b + AHEAD)
        drain_one(b)
        buf = (2 * b) & (NBUF - 1)
        bl = b & (CHUNK - 1)
        half = (b >> LOGC) & 1

        @pl.when((bl == 0) & (b >= 2 * CHUNK))
        def _():
            pltpu.make_async_copy(
                outbuf.at[half], z_hbm.at[pl.ds(base, CHUNK)], osem).wait()

        nj = HIDDEN // (2 * LANES)
        accs = [jnp.zeros((LANES,), jnp.float32) for _ in range(HIDDEN // LANES)]
        bb = jnp.full((LANES,), b, jnp.int32)
        for k in range(TOPK):
            wbits = plsc.load_gather(
                nbwv_v, [bb, jnp.full((LANES,), TOPK + k, jnp.int32)])
            w = plsc.bitcast(wbits, jnp.float32)
            for j in range(nj):
                v = rows_v[buf + k // HALFK, k % HALFK, pl.ds(j * LANES, LANES)]
                lo = plsc.bitcast(jax.lax.shift_left(v, 16), jnp.float32)
                hi = plsc.bitcast(v, jnp.float32)
                accs[j] = accs[j] + w * lo
                accs[nj + j] = accs[nj + j] + w * hi
        for j in range(HIDDEN // LANES):
            outbuf[half, bl, pl.ds(j * LANES, LANES)] = accs[j]

        @pl.when(bl == CHUNK - 1)
        def _():
            start = pl.multiple_of(base + (b - (CHUNK - 1)), CHUNK)
            pltpu.async_copy(outbuf.at[half], z_hbm.at[pl.ds(start, CHUNK)], osem)
        return carry

    lax.fori_loop(0, BPW, item, 0)
    for _ in range(2):
        pltpu.make_async_copy(
            outbuf.at[0], z_hbm.at[pl.ds(base, CHUNK)], osem).wait()


_sc_agg = functools.partial(
    pl.kernel,
    out_type=jax.ShapeDtypeStruct((BATCH, HIDDEN), jnp.float32),
    mesh=plsc.VectorSubcoreMesh(
        core_axis_name="c", subcore_axis_name="s",
        num_cores=NUM_CORES, num_subcores=NUM_SUBCORES),
    scratch_types=[
        pltpu.VMEM((BPW // 128, 128), jnp.int32),
        pltpu.VMEM((BPW, 128), jnp.int32),
        pltpu.VMEM((NBUF, TOPK // 2, HIDDEN // 2), jnp.int32),
        pltpu.VMEM((2, CHUNK, HIDDEN), jnp.float32),
        pltpu.SemaphoreType.DMA,
        pltpu.SemaphoreType.DMA,
        pltpu.SemaphoreType.DMA,
    ],
    compiler_params=pltpu.CompilerParams(needs_layout_passes=False),
)(_sc_agg_body)


def _mlp2_body(z_ref, w_ref, o_ref):
    logits = jax.lax.dot_general(
        w_ref[...], z_ref[...], (((0,), (1,)), ((), ())),
        preferred_element_type=jnp.float32)
    m = jnp.max(logits, axis=0, keepdims=True)
    e = jnp.exp(logits - m)
    s = jnp.sum(e, axis=0, keepdims=True)
    o_ref[...] = (logits - m) - jnp.log(s)


def _mlp2(z, w1):
    return pl.pallas_call(
        _mlp2_body,
        grid=(BATCH // ZROWB,),
        in_specs=[
            pl.BlockSpec((ZROWB, HIDDEN), lambda i: (i, 0)),
            pl.BlockSpec((HIDDEN, N_CLASSES), lambda i: (0, 0)),
        ],
        out_specs=pl.BlockSpec((N_CLASSES, ZROWB), lambda i: (0, i)),
        out_shape=jax.ShapeDtypeStruct((N_CLASSES, BATCH), jnp.float32),
    )(z, w1)


def kernel(x, batch, ppr_idx, ppr_val, W0, W1):
    batch2d = batch.astype(jnp.int32).reshape(BATCH // 128, 128)
    h, comb = _mlp1(x, W0, ppr_idx.astype(jnp.int32).T, ppr_val.T)
    z = _sc_agg(h, batch2d, comb)
    return _mlp2(z, W1).T


# speedup vs baseline: 18.9007x; 1.1036x over previous
import functools

import jax
import jax.numpy as jnp
import numpy as np
from jax import lax
from jax.experimental import pallas as pl
from jax.experimental.pallas import tpu as pltpu
from jax.experimental.pallas import tpu_sc as plsc

N_NODES = 100000
D_FEAT = 128
HIDDEN = 256
N_CLASSES = 349
BATCH = 16384
TOPK = 32

NUM_CORES = 2
NUM_SUBCORES = 16
LANES = 16
NW = NUM_CORES * NUM_SUBCORES
BPW = BATCH // NW
CHUNK = 32
LOGC = 5
ROWB = 2048
ZROWB = 1024


def _mlp1_body(x_ref, w_ref, idxt_ref, valt_ref, o_ref, comb_ref):
    xb = x_ref[...].astype(jnp.bfloat16)
    wb = w_ref[...].astype(jnp.bfloat16)
    h = jnp.dot(xb, wb, preferred_element_type=jnp.float32)
    h = jnp.maximum(h, 0.0)
    ia = jax.lax.bitcast_convert_type(h[:, : HIDDEN // 2], jnp.int32)
    ib = jax.lax.bitcast_convert_type(h[:, HIDDEN // 2:], jnp.int32)
    lo = jax.lax.shift_right_logical(ia + 0x8000, 16)
    hi = ib & jnp.int32(-65536)
    o_ref[...] = lo | hi
    comb_ref[...] = jnp.concatenate([
        jnp.transpose(idxt_ref[...]),
        jax.lax.bitcast_convert_type(jnp.transpose(valt_ref[...]), jnp.int32),
        jnp.zeros((ROWB, 128 - 2 * TOPK), jnp.int32),
    ], axis=1)


def _mlp1(x, w0, ppr_idx_t, ppr_val_t):
    return pl.pallas_call(
        _mlp1_body,
        grid=(pl.cdiv(N_NODES, ROWB),),
        in_specs=[
            pl.BlockSpec((ROWB, D_FEAT), lambda i: (i, 0)),
            pl.BlockSpec((D_FEAT, HIDDEN), lambda i: (0, 0)),
            pl.BlockSpec((TOPK, ROWB), lambda i: (0, i)),
            pl.BlockSpec((TOPK, ROWB), lambda i: (0, i)),
        ],
        out_specs=[
            pl.BlockSpec((ROWB, HIDDEN // 2), lambda i: (i, 0)),
            pl.BlockSpec((ROWB, 128), lambda i: (i, 0)),
        ],
        out_shape=[
            jax.ShapeDtypeStruct((N_NODES, HIDDEN // 2), jnp.int32),
            jax.ShapeDtypeStruct((N_NODES, 128), jnp.int32),
        ],
    )(x, w0, ppr_idx_t, ppr_val_t)


NBUF = 16


def _sc_agg_body(h_hbm, batch_hbm, comb_hbm, z_hbm,
                 batch_v, nbwv_v, rows_v, outbuf, gsem, hsem, osem):
    wid = lax.axis_index("s") * NUM_CORES + lax.axis_index("c")
    base = wid * BPW

    pltpu.sync_copy(batch_hbm.at[pl.ds(wid * (BPW // 128), BPW // 128)], batch_v)

    cps = []
    for c in range(BPW // 128):
        cps.append(pltpu.async_copy(
            comb_hbm.at[batch_v.at[c]], nbwv_v.at[pl.ds(c * 128, 128)], gsem))
    for cp in cps:
        cp.wait()

    HALFK = TOPK // 2

    def fire(b):
        s0 = (2 * b) & (NBUF - 1)
        pltpu.async_copy(
            h_hbm.at[nbwv_v.at[b, pl.ds(0, HALFK)]], rows_v.at[s0], hsem)
        pltpu.async_copy(
            h_hbm.at[nbwv_v.at[b, pl.ds(HALFK, HALFK)]], rows_v.at[s0 + 1], hsem)

    def drain_one(b):
        s0 = (2 * b) & (NBUF - 1)
        pltpu.make_async_copy(
            h_hbm.at[nbwv_v.at[b, pl.ds(0, HALFK)]], rows_v.at[s0], hsem).wait()
        pltpu.make_async_copy(
            h_hbm.at[nbwv_v.at[b, pl.ds(0, HALFK)]], rows_v.at[s0 + 1], hsem).wait()

    AHEAD = NBUF // 2 - 1
    for p in range(AHEAD):
        fire(jnp.int32(p))

    def item(b, carry):
        @pl.when(b + AHEAD < BPW)
        def _():
            fire(b + AHEAD)
        drain_one(b)
        buf = (2 * b) & (NBUF - 1)
        bl = b & (CHUNK - 1)
        half = (b >> LOGC) & 1

        @pl.when((bl == 0) & (b >= 2 * CHUNK))
        def _():
            pltpu.make_async_copy(
                outbuf.at[half], z_hbm.at[pl.ds(base, CHUNK)], osem).wait()

        nj = HIDDEN // (2 * LANES)
        accs = [jnp.zeros((LANES,), jnp.float32) for _ in range(HIDDEN // LANES)]
        bb = jnp.full((LANES,), b, jnp.int32)
        for k in range(TOPK):
            wbits = plsc.load_gather(
                nbwv_v, [bb, jnp.full((LANES,), TOPK + k, jnp.int32)])
            w = plsc.bitcast(wbits, jnp.float32)
            for j in range(nj):
                v = rows_v[buf + k // HALFK, k % HALFK, pl.ds(j * LANES, LANES)]
                lo = plsc.bitcast(jax.lax.shift_left(v, 16), jnp.float32)
                hi = plsc.bitcast(v, jnp.float32)
                accs[j] = accs[j] + w * lo
                accs[nj + j] = accs[nj + j] + w * hi
        for j in range(HIDDEN // LANES):
            outbuf[half, bl, pl.ds(j * LANES, LANES)] = accs[j]

        @pl.when(bl == CHUNK - 1)
        def _():
            start = pl.multiple_of(base + (b - (CHUNK - 1)), CHUNK)
            pltpu.async_copy(outbuf.at[half], z_hbm.at[pl.ds(start, CHUNK)], osem)
        return carry

    lax.fori_loop(0, BPW, item, 0)
    for _ in range(2):
        pltpu.make_async_copy(
            outbuf.at[0], z_hbm.at[pl.ds(base, CHUNK)], osem).wait()


_sc_agg = functools.partial(
    pl.kernel,
    out_type=jax.ShapeDtypeStruct((BATCH, HIDDEN), jnp.float32),
    mesh=plsc.VectorSubcoreMesh(
        core_axis_name="c", subcore_axis_name="s",
        num_cores=NUM_CORES, num_subcores=NUM_SUBCORES),
    scratch_types=[
        pltpu.VMEM((BPW // 128, 128), jnp.int32),
        pltpu.VMEM((BPW, 128), jnp.int32),
        pltpu.VMEM((NBUF, TOPK // 2, HIDDEN // 2), jnp.int32),
        pltpu.VMEM((2, CHUNK, HIDDEN), jnp.float32),
        pltpu.SemaphoreType.DMA,
        pltpu.SemaphoreType.DMA,
        pltpu.SemaphoreType.DMA,
    ],
    compiler_params=pltpu.CompilerParams(needs_layout_passes=False),
)(_sc_agg_body)


def _mlp2_body(z_ref, w_ref, o_ref):
    logits = jax.lax.dot_general(
        w_ref[...], z_ref[...], (((0,), (1,)), ((), ())),
        preferred_element_type=jnp.float32)
    m = jnp.max(logits, axis=0, keepdims=True)
    e = jnp.exp(logits - m)
    s = jnp.sum(e, axis=0, keepdims=True)
    o_ref[...] = (logits - m) - jnp.log(s)


def _mlp2(z, w1):
    return pl.pallas_call(
        _mlp2_body,
        grid=(BATCH // ZROWB,),
        in_specs=[
            pl.BlockSpec((ZROWB, HIDDEN), lambda i: (i, 0)),
            pl.BlockSpec((HIDDEN, N_CLASSES), lambda i: (0, 0)),
        ],
        out_specs=pl.BlockSpec((N_CLASSES, ZROWB), lambda i: (0, i)),
        out_shape=jax.ShapeDtypeStruct((N_CLASSES, BATCH), jnp.float32),
    )(z, w1)


def kernel(x, batch, ppr_idx, ppr_val, W0, W1):
    batch2d = batch.astype(jnp.int32).reshape(BATCH // 128, 128)
    h, comb = _mlp1(x, W0, ppr_idx.astype(jnp.int32).T, ppr_val.T)
    z = _sc_agg(h, batch2d, comb)
    return _mlp2(z, W1).T


# speedup vs baseline: 19.9967x; 1.0580x over previous
import functools

import jax
import jax.numpy as jnp
import numpy as np
from jax import lax
from jax.experimental import pallas as pl
from jax.experimental.pallas import tpu as pltpu
from jax.experimental.pallas import tpu_sc as plsc

N_NODES = 100000
D_FEAT = 128
HIDDEN = 256
N_CLASSES = 349
BATCH = 16384
TOPK = 32

NUM_CORES = 2
NUM_SUBCORES = 16
LANES = 16
NW = NUM_CORES * NUM_SUBCORES
BPW = BATCH // NW
CHUNK = 32
LOGC = 5
ROWB = 4096
ZROWB = 1024


def _mlp1_body(x_ref, w_ref, idxt_ref, valt_ref, o_ref, comb_ref):
    xb = x_ref[...].astype(jnp.bfloat16)
    wb = w_ref[...].astype(jnp.bfloat16)
    h = jnp.dot(xb, wb, preferred_element_type=jnp.float32)
    h = jnp.maximum(h, 0.0)
    ia = jax.lax.bitcast_convert_type(h[:, : HIDDEN // 2], jnp.int32)
    ib = jax.lax.bitcast_convert_type(h[:, HIDDEN // 2:], jnp.int32)
    lo = jax.lax.shift_right_logical(ia + 0x8000, 16)
    hi = ib & jnp.int32(-65536)
    o_ref[...] = lo | hi
    comb_ref[...] = jnp.concatenate([
        jnp.transpose(idxt_ref[...]),
        jax.lax.bitcast_convert_type(jnp.transpose(valt_ref[...]), jnp.int32),
        jnp.zeros((ROWB, 128 - 2 * TOPK), jnp.int32),
    ], axis=1)


def _mlp1(x, w0, ppr_idx_t, ppr_val_t):
    return pl.pallas_call(
        _mlp1_body,
        grid=(pl.cdiv(N_NODES, ROWB),),
        in_specs=[
            pl.BlockSpec((ROWB, D_FEAT), lambda i: (i, 0)),
            pl.BlockSpec((D_FEAT, HIDDEN), lambda i: (0, 0)),
            pl.BlockSpec((TOPK, ROWB), lambda i: (0, i)),
            pl.BlockSpec((TOPK, ROWB), lambda i: (0, i)),
        ],
        out_specs=[
            pl.BlockSpec((ROWB, HIDDEN // 2), lambda i: (i, 0)),
            pl.BlockSpec((ROWB, 128), lambda i: (i, 0)),
        ],
        out_shape=[
            jax.ShapeDtypeStruct((N_NODES, HIDDEN // 2), jnp.int32),
            jax.ShapeDtypeStruct((N_NODES, 128), jnp.int32),
        ],
    )(x, w0, ppr_idx_t, ppr_val_t)


NBUF = 16


def _sc_agg_body(h_hbm, batch_hbm, comb_hbm, z_hbm,
                 batch_v, nbwv_v, rows_v, outbuf, gsem, hsem, osem):
    wid = lax.axis_index("s") * NUM_CORES + lax.axis_index("c")
    base = wid * BPW

    pltpu.sync_copy(batch_hbm.at[pl.ds(wid * (BPW // 128), BPW // 128)], batch_v)

    cps = []
    for c in range(BPW // 128):
        cps.append(pltpu.async_copy(
            comb_hbm.at[batch_v.at[c]], nbwv_v.at[pl.ds(c * 128, 128)], gsem))
    for cp in cps:
        cp.wait()

    HALFK = TOPK // 2

    def fire(b):
        s0 = (2 * b) & (NBUF - 1)
        pltpu.async_copy(
            h_hbm.at[nbwv_v.at[b, pl.ds(0, HALFK)]], rows_v.at[s0], hsem)
        pltpu.async_copy(
            h_hbm.at[nbwv_v.at[b, pl.ds(HALFK, HALFK)]], rows_v.at[s0 + 1], hsem)

    def drain_one(b):
        s0 = (2 * b) & (NBUF - 1)
        pltpu.make_async_copy(
            h_hbm.at[nbwv_v.at[b, pl.ds(0, HALFK)]], rows_v.at[s0], hsem).wait()
        pltpu.make_async_copy(
            h_hbm.at[nbwv_v.at[b, pl.ds(0, HALFK)]], rows_v.at[s0 + 1], hsem).wait()

    AHEAD = NBUF // 2 - 1
    for p in range(AHEAD):
        fire(jnp.int32(p))

    def item(b, carry):
        @pl.when(b + AHEAD < BPW)
        def _():
            fire(b + AHEAD)
        drain_one(b)
        buf = (2 * b) & (NBUF - 1)
        bl = b & (CHUNK - 1)
        half = (b >> LOGC) & 1

        @pl.when((bl == 0) & (b >= 2 * CHUNK))
        def _():
            pltpu.make_async_copy(
                outbuf.at[half], z_hbm.at[pl.ds(base, CHUNK)], osem).wait()

        nj = HIDDEN // (2 * LANES)
        accs = [jnp.zeros((LANES,), jnp.float32) for _ in range(HIDDEN // LANES)]
        bb = jnp.full((LANES,), b, jnp.int32)
        for k in range(TOPK):
            wbits = plsc.load_gather(
                nbwv_v, [bb, jnp.full((LANES,), TOPK + k, jnp.int32)])
            w = plsc.bitcast(wbits, jnp.float32)
            for j in range(nj):
                v = rows_v[buf + k // HALFK, k % HALFK, pl.ds(j * LANES, LANES)]
                lo = plsc.bitcast(jax.lax.shift_left(v, 16), jnp.float32)
                hi = plsc.bitcast(v, jnp.float32)
                accs[j] = accs[j] + w * lo
                accs[nj + j] = accs[nj + j] + w * hi
        for j in range(HIDDEN // LANES):
            outbuf[half, bl, pl.ds(j * LANES, LANES)] = accs[j]

        @pl.when(bl == CHUNK - 1)
        def _():
            start = pl.multiple_of(base + (b - (CHUNK - 1)), CHUNK)
            pltpu.async_copy(outbuf.at[half], z_hbm.at[pl.ds(start, CHUNK)], osem)
        return carry

    lax.fori_loop(0, BPW, item, 0)
    for _ in range(2):
        pltpu.make_async_copy(
            outbuf.at[0], z_hbm.at[pl.ds(base, CHUNK)], osem).wait()


_sc_agg = functools.partial(
    pl.kernel,
    out_type=jax.ShapeDtypeStruct((BATCH, HIDDEN), jnp.float32),
    mesh=plsc.VectorSubcoreMesh(
        core_axis_name="c", subcore_axis_name="s",
        num_cores=NUM_CORES, num_subcores=NUM_SUBCORES),
    scratch_types=[
        pltpu.VMEM((BPW // 128, 128), jnp.int32),
        pltpu.VMEM((BPW, 128), jnp.int32),
        pltpu.VMEM((NBUF, TOPK // 2, HIDDEN // 2), jnp.int32),
        pltpu.VMEM((2, CHUNK, HIDDEN), jnp.float32),
        pltpu.SemaphoreType.DMA,
        pltpu.SemaphoreType.DMA,
        pltpu.SemaphoreType.DMA,
    ],
    compiler_params=pltpu.CompilerParams(needs_layout_passes=False),
)(_sc_agg_body)


def _mlp2_body(z_ref, w_ref, o_ref):
    logits = jax.lax.dot_general(
        w_ref[...], z_ref[...], (((0,), (1,)), ((), ())),
        preferred_element_type=jnp.float32)
    m = jnp.max(logits, axis=0, keepdims=True)
    e = jnp.exp(logits - m)
    s = jnp.sum(e, axis=0, keepdims=True)
    o_ref[...] = (logits - m) - jnp.log(s)


def _mlp2(z, w1):
    return pl.pallas_call(
        _mlp2_body,
        grid=(BATCH // ZROWB,),
        in_specs=[
            pl.BlockSpec((ZROWB, HIDDEN), lambda i: (i, 0)),
            pl.BlockSpec((HIDDEN, N_CLASSES), lambda i: (0, 0)),
        ],
        out_specs=pl.BlockSpec((N_CLASSES, ZROWB), lambda i: (0, i)),
        out_shape=jax.ShapeDtypeStruct((N_CLASSES, BATCH), jnp.float32),
    )(z, w1)


def kernel(x, batch, ppr_idx, ppr_val, W0, W1):
    batch2d = batch.astype(jnp.int32).reshape(BATCH // 128, 128)
    h, comb = _mlp1(x, W0, ppr_idx.astype(jnp.int32).T, ppr_val.T)
    z = _sc_agg(h, batch2d, comb)
    return _mlp2(z, W1).T


# speedup vs baseline: 20.4074x; 1.0205x over previous
import functools

import jax
import jax.numpy as jnp
import numpy as np
from jax import lax
from jax.experimental import pallas as pl
from jax.experimental.pallas import tpu as pltpu
from jax.experimental.pallas import tpu_sc as plsc

N_NODES = 100000
D_FEAT = 128
HIDDEN = 256
N_CLASSES = 349
BATCH = 16384
TOPK = 32

NUM_CORES = 2
NUM_SUBCORES = 16
LANES = 16
NW = NUM_CORES * NUM_SUBCORES
BPW = BATCH // NW
CHUNK = 32
LOGC = 5
ROWB = 8192
ZROWB = 1024


def _mlp1_body(x_ref, w_ref, idxt_ref, valt_ref, o_ref, comb_ref):
    xb = x_ref[...].astype(jnp.bfloat16)
    wb = w_ref[...].astype(jnp.bfloat16)
    h = jnp.dot(xb, wb, preferred_element_type=jnp.float32)
    h = jnp.maximum(h, 0.0)
    ia = jax.lax.bitcast_convert_type(h[:, : HIDDEN // 2], jnp.int32)
    ib = jax.lax.bitcast_convert_type(h[:, HIDDEN // 2:], jnp.int32)
    lo = jax.lax.shift_right_logical(ia + 0x8000, 16)
    hi = ib & jnp.int32(-65536)
    o_ref[...] = lo | hi
    comb_ref[...] = jnp.concatenate([
        jnp.transpose(idxt_ref[...]),
        jax.lax.bitcast_convert_type(jnp.transpose(valt_ref[...]), jnp.int32),
        jnp.zeros((ROWB, 128 - 2 * TOPK), jnp.int32),
    ], axis=1)


def _mlp1(x, w0, ppr_idx_t, ppr_val_t):
    return pl.pallas_call(
        _mlp1_body,
        grid=(pl.cdiv(N_NODES, ROWB),),
        in_specs=[
            pl.BlockSpec((ROWB, D_FEAT), lambda i: (i, 0)),
            pl.BlockSpec((D_FEAT, HIDDEN), lambda i: (0, 0)),
            pl.BlockSpec((TOPK, ROWB), lambda i: (0, i)),
            pl.BlockSpec((TOPK, ROWB), lambda i: (0, i)),
        ],
        out_specs=[
            pl.BlockSpec((ROWB, HIDDEN // 2), lambda i: (i, 0)),
            pl.BlockSpec((ROWB, 128), lambda i: (i, 0)),
        ],
        out_shape=[
            jax.ShapeDtypeStruct((N_NODES, HIDDEN // 2), jnp.int32),
            jax.ShapeDtypeStruct((N_NODES, 128), jnp.int32),
        ],
    )(x, w0, ppr_idx_t, ppr_val_t)


NBUF = 16


def _sc_agg_body(h_hbm, batch_hbm, comb_hbm, z_hbm,
                 batch_v, nbwv_v, rows_v, outbuf, gsem, hsem, osem):
    wid = lax.axis_index("s") * NUM_CORES + lax.axis_index("c")
    base = wid * BPW

    pltpu.sync_copy(batch_hbm.at[pl.ds(wid * (BPW // 128), BPW // 128)], batch_v)

    cps = []
    for c in range(BPW // 128):
        cps.append(pltpu.async_copy(
            comb_hbm.at[batch_v.at[c]], nbwv_v.at[pl.ds(c * 128, 128)], gsem))
    for cp in cps:
        cp.wait()

    HALFK = TOPK // 2

    def fire(b):
        s0 = (2 * b) & (NBUF - 1)
        pltpu.async_copy(
            h_hbm.at[nbwv_v.at[b, pl.ds(0, HALFK)]], rows_v.at[s0], hsem)
        pltpu.async_copy(
            h_hbm.at[nbwv_v.at[b, pl.ds(HALFK, HALFK)]], rows_v.at[s0 + 1], hsem)

    def drain_one(b):
        s0 = (2 * b) & (NBUF - 1)
        pltpu.make_async_copy(
            h_hbm.at[nbwv_v.at[b, pl.ds(0, HALFK)]], rows_v.at[s0], hsem).wait()
        pltpu.make_async_copy(
            h_hbm.at[nbwv_v.at[b, pl.ds(0, HALFK)]], rows_v.at[s0 + 1], hsem).wait()

    AHEAD = NBUF // 2 - 1
    for p in range(AHEAD):
        fire(jnp.int32(p))

    def item(b, carry):
        @pl.when(b + AHEAD < BPW)
        def _():
            fire(b + AHEAD)
        drain_one(b)
        buf = (2 * b) & (NBUF - 1)
        bl = b & (CHUNK - 1)
        half = (b >> LOGC) & 1

        @pl.when((bl == 0) & (b >= 2 * CHUNK))
        def _():
            pltpu.make_async_copy(
                outbuf.at[half], z_hbm.at[pl.ds(base, CHUNK)], osem).wait()

        nj = HIDDEN // (2 * LANES)
        accs = [jnp.zeros((LANES,), jnp.float32) for _ in range(HIDDEN // LANES)]
        bb = jnp.full((LANES,), b, jnp.int32)
        for k in range(TOPK):
            wbits = plsc.load_gather(
                nbwv_v, [bb, jnp.full((LANES,), TOPK + k, jnp.int32)])
            w = plsc.bitcast(wbits, jnp.float32)
            for j in range(nj):
                v = rows_v[buf + k // HALFK, k % HALFK, pl.ds(j * LANES, LANES)]
                lo = plsc.bitcast(jax.lax.shift_left(v, 16), jnp.float32)
                hi = plsc.bitcast(v, jnp.float32)
                accs[j] = accs[j] + w * lo
                accs[nj + j] = accs[nj + j] + w * hi
        for j in range(HIDDEN // LANES):
            outbuf[half, bl, pl.ds(j * LANES, LANES)] = accs[j]

        @pl.when(bl == CHUNK - 1)
        def _():
            start = pl.multiple_of(base + (b - (CHUNK - 1)), CHUNK)
            pltpu.async_copy(outbuf.at[half], z_hbm.at[pl.ds(start, CHUNK)], osem)
        return carry

    lax.fori_loop(0, BPW, item, 0)
    for _ in range(2):
        pltpu.make_async_copy(
            outbuf.at[0], z_hbm.at[pl.ds(base, CHUNK)], osem).wait()


_sc_agg = functools.partial(
    pl.kernel,
    out_type=jax.ShapeDtypeStruct((BATCH, HIDDEN), jnp.float32),
    mesh=plsc.VectorSubcoreMesh(
        core_axis_name="c", subcore_axis_name="s",
        num_cores=NUM_CORES, num_subcores=NUM_SUBCORES),
    scratch_types=[
        pltpu.VMEM((BPW // 128, 128), jnp.int32),
        pltpu.VMEM((BPW, 128), jnp.int32),
        pltpu.VMEM((NBUF, TOPK // 2, HIDDEN // 2), jnp.int32),
        pltpu.VMEM((2, CHUNK, HIDDEN), jnp.float32),
        pltpu.SemaphoreType.DMA,
        pltpu.SemaphoreType.DMA,
        pltpu.SemaphoreType.DMA,
    ],
    compiler_params=pltpu.CompilerParams(needs_layout_passes=False),
)(_sc_agg_body)


def _mlp2_body(z_ref, w_ref, o_ref):
    logits = jax.lax.dot_general(
        w_ref[...], z_ref[...], (((0,), (1,)), ((), ())),
        preferred_element_type=jnp.float32)
    m = jnp.max(logits, axis=0, keepdims=True)
    e = jnp.exp(logits - m)
    s = jnp.sum(e, axis=0, keepdims=True)
    o_ref[...] = (logits - m) - jnp.log(s)


def _mlp2(z, w1):
    return pl.pallas_call(
        _mlp2_body,
        grid=(BATCH // ZROWB,),
        in_specs=[
            pl.BlockSpec((ZROWB, HIDDEN), lambda i: (i, 0)),
            pl.BlockSpec((HIDDEN, N_CLASSES), lambda i: (0, 0)),
        ],
        out_specs=pl.BlockSpec((N_CLASSES, ZROWB), lambda i: (0, i)),
        out_shape=jax.ShapeDtypeStruct((N_CLASSES, BATCH), jnp.float32),
    )(z, w1)


def kernel(x, batch, ppr_idx, ppr_val, W0, W1):
    batch2d = batch.astype(jnp.int32).reshape(BATCH // 128, 128)
    h, comb = _mlp1(x, W0, ppr_idx.astype(jnp.int32).T, ppr_val.T)
    z = _sc_agg(h, batch2d, comb)
    return _mlp2(z, W1).T
